# Initial kernel scaffold; baseline (speedup 1.0000x reference)
import jax
import jax.numpy as jnp
from jax.experimental import pallas as pl


def kernel(query, g_prompt, e_prompt_pool, e_prompt_keys):
    raise NotImplementedError("write your pallas kernel here")


# speedup vs baseline: 2.0952x; 2.0952x over previous
import jax
import jax.numpy as jnp
from jax import lax
from jax.experimental import pallas as pl
from jax.experimental.pallas import tpu as pltpu
from jax.experimental.pallas import tpu_sc as plsc

_NC = 2
_NS = 16
_NW = _NC * _NS
_TOPK = 8


def _topk_body(q_ref, kt_ref, idx_ref):
    q = q_ref[...]
    qs = jnp.sum(q * q, axis=1, keepdims=True)
    qn = q / jnp.maximum(jnp.sqrt(qs), 1e-12)
    kt = kt_ref[...]
    ss = jnp.sum(kt * kt, axis=0, keepdims=True)
    kn = kt / jnp.maximum(jnp.sqrt(ss), 1e-12)
    s = lax.dot_general(
        qn, kn, (((1,), (0,)), ((), ())),
        preferred_element_type=jnp.float32,
    )
    b, p = s.shape
    e_len = idx_ref.shape[1] // _TOPK
    iota = lax.broadcasted_iota(jnp.int32, (b, p), 1)
    sub = lax.broadcasted_iota(jnp.int32, (b, e_len), 1)
    cur = s
    for j in range(_TOPK):
        m = jnp.max(cur, axis=1, keepdims=True)
        sel = jnp.where(cur == m, iota, p)
        idx_j = jnp.min(sel, axis=1)
        idx_ref[:, pl.ds(j * e_len, e_len)] = idx_j[:, None] * e_len + sub
        cur = jnp.where(iota == idx_j[:, None], -jnp.inf, cur)


def _sc_gather_body(pool_hbm, g_hbm, idx_hbm, out_hbm, idx_v, buf, sem):
    rows_per = idx_v.shape[0]
    g_len = g_hbm.shape[0]
    n_e = idx_v.shape[1]
    base = (lax.axis_index("s") * _NC + lax.axis_index("c")) * rows_per
    pltpu.sync_copy(idx_hbm.at[pl.ds(base, rows_per)], idx_v)
    pltpu.sync_copy(g_hbm, buf.at[pl.ds(0, g_len)])
    def body(i, carry):
        r = base + i
        dst = buf.at[pl.ds(g_len, n_e)]
        pltpu.async_copy(pool_hbm.at[idx_v.at[i]], dst, sem).wait()
        pltpu.sync_copy(buf, out_hbm.at[r])
        return carry
    lax.fori_loop(0, rows_per, body, 0)


def kernel(query, g_prompt, e_prompt_pool, e_prompt_keys):
    b, d = query.shape
    pool, e_len, _ = e_prompt_pool.shape
    g_len = g_prompt.shape[1]
    n_e = _TOPK * e_len

    idx = pl.pallas_call(
        _topk_body,
        out_shape=jax.ShapeDtypeStruct((b, n_e), jnp.int32),
    )(query, e_prompt_keys.T)

    pool_rows = e_prompt_pool.reshape(pool * e_len, d)
    g_rows = g_prompt.reshape(g_len, d)

    rows_per = b // _NW
    sc = pl.kernel(
        _sc_gather_body,
        out_type=jax.ShapeDtypeStruct((b, g_len + n_e, d), jnp.float32),
        mesh=plsc.VectorSubcoreMesh(core_axis_name="c", subcore_axis_name="s"),
        scratch_types=[
            pltpu.VMEM((rows_per, n_e), jnp.int32),
            pltpu.VMEM((g_len + n_e, d), jnp.float32),
            pltpu.SemaphoreType.DMA,
        ],
    )
    return sc(pool_rows, g_rows, idx)


# speedup vs baseline: 2.1621x; 1.0320x over previous
import jax
import jax.numpy as jnp
from jax import lax
from jax.experimental import pallas as pl
from jax.experimental.pallas import tpu as pltpu
from jax.experimental.pallas import tpu_sc as plsc

_NC = 2
_NS = 16
_NW = _NC * _NS
_TOPK = 8


def _topk_body(q_ref, kt_ref, idx_ref):
    q = q_ref[...]
    qs = jnp.sum(q * q, axis=1, keepdims=True)
    qn = q / jnp.maximum(jnp.sqrt(qs), 1e-12)
    kt = kt_ref[...]
    ss = jnp.sum(kt * kt, axis=0, keepdims=True)
    kn = kt / jnp.maximum(jnp.sqrt(ss), 1e-12)
    s = lax.dot_general(
        qn, kn, (((1,), (0,)), ((), ())),
        preferred_element_type=jnp.float32,
    )
    b, p = s.shape
    e_len = idx_ref.shape[1] // _TOPK
    iota = lax.broadcasted_iota(jnp.int32, (b, p), 1)
    sub = lax.broadcasted_iota(jnp.int32, (b, e_len), 1)
    cur = s
    for j in range(_TOPK):
        m = jnp.max(cur, axis=1, keepdims=True)
        sel = jnp.where(cur == m, iota, p)
        idx_j = jnp.min(sel, axis=1)
        idx_ref[:, pl.ds(j * e_len, e_len)] = idx_j[:, None] * e_len + sub
        cur = jnp.where(iota == idx_j[:, None], -jnp.inf, cur)


def _sc_gather_body(pool_hbm, g_hbm, idx_hbm, out_hbm, idx_v, buf0, buf1,
                    sem0, sem1):
    rows_per = idx_v.shape[0]
    g_len = g_hbm.shape[0]
    n_e = idx_v.shape[1]
    base = (lax.axis_index("s") * _NC + lax.axis_index("c")) * rows_per
    pltpu.sync_copy(idx_hbm.at[pl.ds(base, rows_per)], idx_v)
    pltpu.sync_copy(g_hbm, buf0.at[pl.ds(0, g_len)])
    pltpu.sync_copy(g_hbm, buf1.at[pl.ds(0, g_len)])

    def start(i, buf, sem):
        pltpu.async_copy(pool_hbm.at[idx_v.at[i]], buf.at[pl.ds(g_len, n_e)],
                         sem)

    def wait(i, buf, sem):
        pltpu.make_async_copy(pool_hbm.at[idx_v.at[i]],
                              buf.at[pl.ds(g_len, n_e)], sem).wait()

    start(0, buf0, sem0)
    def body(j, carry):
        i = 2 * j
        wait(i, buf0, sem0)
        start(i + 1, buf1, sem1)
        pltpu.sync_copy(buf0, out_hbm.at[base + i])
        wait(i + 1, buf1, sem1)
        @pl.when(i + 2 < rows_per)
        def _():
            start(i + 2, buf0, sem0)
        pltpu.sync_copy(buf1, out_hbm.at[base + i + 1])
        return carry
    lax.fori_loop(0, rows_per // 2, body, 0)


def kernel(query, g_prompt, e_prompt_pool, e_prompt_keys):
    b, d = query.shape
    pool, e_len, _ = e_prompt_pool.shape
    g_len = g_prompt.shape[1]
    n_e = _TOPK * e_len

    idx = pl.pallas_call(
        _topk_body,
        out_shape=jax.ShapeDtypeStruct((b, n_e), jnp.int32),
    )(query, e_prompt_keys.T)

    pool_rows = e_prompt_pool.reshape(pool * e_len, d)
    g_rows = g_prompt.reshape(g_len, d)

    rows_per = b // _NW
    sc = pl.kernel(
        _sc_gather_body,
        out_type=jax.ShapeDtypeStruct((b, g_len + n_e, d), jnp.float32),
        mesh=plsc.VectorSubcoreMesh(core_axis_name="c", subcore_axis_name="s"),
        scratch_types=[
            pltpu.VMEM((rows_per, n_e), jnp.int32),
            pltpu.VMEM((g_len + n_e, d), jnp.float32),
            pltpu.VMEM((g_len + n_e, d), jnp.float32),
            pltpu.SemaphoreType.DMA,
            pltpu.SemaphoreType.DMA,
        ],
    )
    return sc(pool_rows, g_rows, idx)
